# Initial kernel scaffold; baseline (speedup 1.0000x reference)
import jax
import jax.numpy as jnp
from jax.experimental import pallas as pl


def kernel(node_token_idx, edge_index, e1, rel, e2_multi, word_emb, W_msg, Wi, Wh, bi, bh, bn_gamma, bn_beta, rel_emb):
    raise NotImplementedError("write your pallas kernel here")


# speedup vs baseline: 1.8930x; 1.8930x over previous
import functools

import jax
import jax.numpy as jnp
from jax import lax
from jax.experimental import pallas as pl
from jax.experimental.pallas import tpu as pltpu
from jax.experimental.pallas import tpu_sc as plsc

N = 10000
E = 160000
H = 256
B = 1024
R = 64
V = 50000
T = 4

NC = 2
NS = 16
NW = NC * NS

NP = 10240
TOK_PAD = NP * T
E_PAD = 163840
ROWS_SC = NP // NS
HH = H // 2
BLK = 1024

F32 = jnp.float32


def _dot_nt(a, b):
    return lax.dot_general(a, b, (((1,), (1,)), ((), ())),
                           preferred_element_type=F32)


def _dot_nn(a, b):
    return lax.dot_general(a, b, (((1,), (0,)), ((), ())),
                           preferred_element_type=F32)


@functools.lru_cache(maxsize=None)
def _mesh():
    return plsc.VectorSubcoreMesh(core_axis_name="c", subcore_axis_name="s",
                                  num_cores=NC, num_subcores=NS)


def _k1_body(tok_idx, wemb, out, idx_v, rows_v, sem):
    wid = lax.axis_index("s") * NC + lax.axis_index("c")
    per = TOK_PAD // NW
    base = wid * per

    def chunk(j, carry):
        off = base + j * 128
        pltpu.sync_copy(tok_idx.at[pl.ds(off, 128)], idx_v)
        pltpu.async_copy(wemb.at[idx_v], rows_v, sem).wait()
        pltpu.sync_copy(rows_v, out.at[pl.ds(off, 128)])
        return carry

    lax.fori_loop(0, per // 128, chunk, 0)


@functools.lru_cache(maxsize=None)
def _k1_kernel():
    return pl.kernel(
        _k1_body,
        out_type=jax.ShapeDtypeStruct((TOK_PAD, H), F32),
        mesh=_mesh(),
        scratch_types=[
            pltpu.VMEM((128,), jnp.int32),
            pltpu.VMEM((128, H), F32),
            pltpu.SemaphoreType.DMA,
        ],
    )


def _k1_call(tok_idx, wemb):
    return _k1_kernel()(tok_idx, wemb)


def _k2_body(tok2, wmsg, wh, bh, nf_o, x1a_o, x1b_o, gh_o):
    t = tok2[...]
    nf = (t[:, 0:H] + t[:, H:2 * H] + t[:, 2 * H:3 * H] + t[:, 3 * H:4 * H]) * 0.25
    x1 = _dot_nn(nf, wmsg[...])
    x1a_o[...] = x1[:, :HH]
    x1b_o[...] = x1[:, HH:]
    gh_o[...] = _dot_nt(nf, wh[...]) + bh[...]
    nf_o[...] = nf


def _k2_call(tok2, W_msg, Wh, bh_row):
    grid = (NP // BLK,)
    return pl.pallas_call(
        _k2_body,
        grid=grid,
        in_specs=[
            pl.BlockSpec((BLK, 4 * H), lambda i: (i, 0)),
            pl.BlockSpec((H, H), lambda i: (0, 0)),
            pl.BlockSpec((3 * H, H), lambda i: (0, 0)),
            pl.BlockSpec((1, 3 * H), lambda i: (0, 0)),
        ],
        out_specs=[
            pl.BlockSpec((BLK, H), lambda i: (i, 0)),
            pl.BlockSpec((BLK, HH), lambda i: (i, 0)),
            pl.BlockSpec((BLK, HH), lambda i: (i, 0)),
            pl.BlockSpec((BLK, 3 * H), lambda i: (i, 0)),
        ],
        out_shape=[
            jax.ShapeDtypeStruct((NP, H), F32),
            jax.ShapeDtypeStruct((NP, HH), F32),
            jax.ShapeDtypeStruct((NP, HH), F32),
            jax.ShapeDtypeStruct((NP, 3 * H), F32),
        ],
    )(tok2, W_msg, Wh, bh_row)


def _k3_body(src, dst, x1a, x1b, zrows, agg_a, agg_b,
             idx_s, idx_d, rows_v, sem, shared):
    cid = lax.axis_index("c")
    sid = lax.axis_index("s")

    pltpu.sync_copy(zrows.at[pl.ds(sid * ROWS_SC, ROWS_SC)],
                    shared.at[pl.ds(sid * ROWS_SC, ROWS_SC)])
    plsc.subcore_barrier()

    def run(table, out_ref):
        base = sid * (E_PAD // NS)

        def chunk(j, carry):
            off = base + j * 128
            pltpu.sync_copy(src.at[pl.ds(off, 128)], idx_s)
            pltpu.sync_copy(dst.at[pl.ds(off, 128)], idx_d)
            pltpu.async_copy(table.at[idx_s], rows_v, sem).wait()
            pltpu.sync_copy(rows_v, shared.at[idx_d], add=True)
            return carry

        lax.fori_loop(0, (E_PAD // NS) // 128, chunk, 0)
        plsc.subcore_barrier()
        pltpu.sync_copy(shared.at[pl.ds(sid * ROWS_SC, ROWS_SC)],
                        out_ref.at[pl.ds(sid * ROWS_SC, ROWS_SC)])

    @pl.when(cid == 0)
    def _():
        run(x1a, agg_a)

    @pl.when(cid == 1)
    def _():
        run(x1b, agg_b)


@functools.lru_cache(maxsize=None)
def _k3_kernel():
    return pl.kernel(
        _k3_body,
        out_type=[
            jax.ShapeDtypeStruct((NP, HH), F32),
            jax.ShapeDtypeStruct((NP, HH), F32),
        ],
        mesh=_mesh(),
        scratch_types=[
            pltpu.VMEM((128,), jnp.int32),
            pltpu.VMEM((128,), jnp.int32),
            pltpu.VMEM((128, HH), F32),
            pltpu.SemaphoreType.DMA,
            pltpu.VMEM_SHARED((NP, HH), F32),
        ],
    )


def _k3_call(src, dst, x1a, x1b, zrows):
    return _k3_kernel()(src, dst, x1a, x1b, zrows)


def _k4_body(agg_a, agg_b, gh, nf, wi, bi, h_o, sums_o):
    i = pl.program_id(0)
    agg = jnp.concatenate([agg_a[...], agg_b[...]], axis=1)
    gi = _dot_nt(agg, wi[...]) + bi[...]
    ghv = gh[...]
    r = jax.nn.sigmoid(gi[:, 0:H] + ghv[:, 0:H])
    z = jax.nn.sigmoid(gi[:, H:2 * H] + ghv[:, H:2 * H])
    n = jnp.tanh(gi[:, 2 * H:] + r * ghv[:, 2 * H:])
    h = (1.0 - z) * n + z * nf[...]
    h_o[...] = h
    row = lax.broadcasted_iota(jnp.int32, (BLK, 1), 0) + i * BLK
    hm = jnp.where(row < N, h, 0.0)
    s = jnp.sum(hm, axis=0, keepdims=True)
    ss = jnp.sum(hm * hm, axis=0, keepdims=True)
    pack = jnp.concatenate([s, ss, jnp.zeros((6, H), dtype=F32)], axis=0)

    @pl.when(i == 0)
    def _():
        sums_o[...] = pack

    @pl.when(i > 0)
    def _():
        sums_o[...] = sums_o[...] + pack


def _k4_call(agg_a, agg_b, gh, nf, Wi, bi_row):
    grid = (NP // BLK,)
    return pl.pallas_call(
        _k4_body,
        grid=grid,
        in_specs=[
            pl.BlockSpec((BLK, HH), lambda i: (i, 0)),
            pl.BlockSpec((BLK, HH), lambda i: (i, 0)),
            pl.BlockSpec((BLK, 3 * H), lambda i: (i, 0)),
            pl.BlockSpec((BLK, H), lambda i: (i, 0)),
            pl.BlockSpec((3 * H, H), lambda i: (0, 0)),
            pl.BlockSpec((1, 3 * H), lambda i: (0, 0)),
        ],
        out_specs=[
            pl.BlockSpec((BLK, H), lambda i: (i, 0)),
            pl.BlockSpec((8, H), lambda i: (0, 0)),
        ],
        out_shape=[
            jax.ShapeDtypeStruct((NP, H), F32),
            jax.ShapeDtypeStruct((8, H), F32),
        ],
    )(agg_a, agg_b, gh, nf, Wi, bi_row)


def _k5_body(e1_idx, rel_idx, h, rel_emb, he, re, idx_v, rows_v, sem):
    wid = lax.axis_index("s") * NC + lax.axis_index("c")
    per = B // NW
    base = wid * per
    pltpu.sync_copy(e1_idx.at[pl.ds(base, per)], idx_v)
    pltpu.async_copy(h.at[idx_v], rows_v, sem).wait()
    pltpu.sync_copy(rows_v, he.at[pl.ds(base, per)])
    pltpu.sync_copy(rel_idx.at[pl.ds(base, per)], idx_v)
    pltpu.async_copy(rel_emb.at[idx_v], rows_v, sem).wait()
    pltpu.sync_copy(rows_v, re.at[pl.ds(base, per)])


@functools.lru_cache(maxsize=None)
def _k5_kernel():
    return pl.kernel(
        _k5_body,
        out_type=[
            jax.ShapeDtypeStruct((B, H), F32),
            jax.ShapeDtypeStruct((B, H), F32),
        ],
        mesh=_mesh(),
        scratch_types=[
            pltpu.VMEM((B // NW,), jnp.int32),
            pltpu.VMEM((B // NW, H), F32),
            pltpu.SemaphoreType.DMA,
        ],
    )


def _k5_call(e1_idx, rel_idx, h, rel_emb):
    return _k5_kernel()(e1_idx, rel_idx, h, rel_emb)


def _k6_body(he, re, sums, gamma, beta, h, e2, logits_o, loss_o):
    i = pl.program_id(0)
    ng = pl.num_programs(0)
    inv_n = 1.0 / N
    mean = sums[0:1, :] * inv_n
    var = sums[1:2, :] * inv_n - mean * mean
    sc = lax.rsqrt(var + 1e-5) * gamma[...]
    q = ((he[...] - mean) * sc + beta[...]) * re[...]
    hb = (h[...] - mean) * sc + beta[...]
    lg = jax.nn.sigmoid(_dot_nt(q, hb))
    logits_o[...] = lg
    p = jnp.clip(lg, 1e-7, 1.0 - 1e-7)
    e2v = e2[...]
    col = lax.broadcasted_iota(jnp.int32, (1, BLK), 1) + i * BLK
    term = e2v * jnp.log(p) + (1.0 - e2v) * jnp.log(1.0 - p)
    part = jnp.sum(jnp.where(col < N, term, 0.0))

    @pl.when(i == 0)
    def _():
        loss_o[0, 0] = part

    @pl.when(i > 0)
    def _():
        loss_o[0, 0] = loss_o[0, 0] + part

    @pl.when(i == ng - 1)
    def _():
        loss_o[0, 0] = loss_o[0, 0] * (-1.0 / (B * N))


def _k6_call(he, re, sums, gamma_row, beta_row, h, e2_pad):
    grid = (NP // BLK,)
    return pl.pallas_call(
        _k6_body,
        grid=grid,
        in_specs=[
            pl.BlockSpec((B, H), lambda i: (0, 0)),
            pl.BlockSpec((B, H), lambda i: (0, 0)),
            pl.BlockSpec((8, H), lambda i: (0, 0)),
            pl.BlockSpec((1, H), lambda i: (0, 0)),
            pl.BlockSpec((1, H), lambda i: (0, 0)),
            pl.BlockSpec((BLK, H), lambda i: (i, 0)),
            pl.BlockSpec((B, BLK), lambda i: (0, i)),
        ],
        out_specs=[
            pl.BlockSpec((B, BLK), lambda i: (0, i)),
            pl.BlockSpec(memory_space=pltpu.SMEM),
        ],
        out_shape=[
            jax.ShapeDtypeStruct((B, NP), F32),
            jax.ShapeDtypeStruct((1, 1), F32),
        ],
    )(he, re, sums, gamma_row, beta_row, h, e2_pad)


def kernel(node_token_idx, edge_index, e1, rel, e2_multi, word_emb,
           W_msg, Wi, Wh, bi, bh, bn_gamma, bn_beta, rel_emb):
    tok_flat = jnp.concatenate(
        [node_token_idx.reshape(-1),
         jnp.zeros((TOK_PAD - N * T,), jnp.int32)])
    src = jnp.concatenate(
        [edge_index[0], jnp.zeros((E_PAD - E,), jnp.int32)])
    dst = jnp.concatenate(
        [edge_index[1], jnp.full((E_PAD - E,), N, jnp.int32)])
    e2_pad = jnp.concatenate(
        [e2_multi, jnp.zeros((B, NP - N), F32)], axis=1)

    tok_rows = _k1_call(tok_flat, word_emb)
    tok2 = tok_rows.reshape(NP, 4 * H)

    nf, x1a, x1b, gh = _k2_call(tok2, W_msg, Wh, bh.reshape(1, 3 * H))

    zrows = jnp.zeros((NP, HH), F32)
    agg_a, agg_b = _k3_call(src, dst, x1a, x1b, zrows)

    h, sums = _k4_call(agg_a, agg_b, gh, nf, Wi, bi.reshape(1, 3 * H))

    he, re = _k5_call(e1[:, 0], rel[:, 0], h, rel_emb)

    logits_pad, loss = _k6_call(he, re, sums, bn_gamma.reshape(1, H),
                                bn_beta.reshape(1, H), h, e2_pad)
    return logits_pad[:, :N], loss[0, 0]
